# baseline (speedup 1.0000x reference)
import functools

import jax
import jax.numpy as jnp
from jax import lax
from jax.experimental import pallas as pl
from jax.experimental.pallas import tpu as pltpu
from jax.experimental.pallas import tpu_sc as plsc

N = 10000
E = 160000
L = 4
H = 8
C = 16
D = 128
OUT_DIM = 1000

NPAD = 10240
EF = E + N
EPAD = 172032
BPAD = 12288
CH = 128
NWORK = 32
BM = 1024
CB = 512
PW = 256

_f32 = jnp.float32
_SDS = jax.ShapeDtypeStruct


def _elu(v):
    return jnp.where(v > 0, v, jnp.exp(jnp.minimum(v, 0.0)) - 1.0)


def _mm_act(x, w, b, act):
    M, K = x.shape
    Nn = w.shape[1]

    def body(x_ref, w_ref, b_ref, o_ref):
        acc = jnp.dot(x_ref[...], w_ref[...], preferred_element_type=_f32)
        acc = acc + b_ref[...]
        o_ref[...] = _elu(acc) if act else acc

    return pl.pallas_call(
        body,
        grid=(M // BM,),
        in_specs=[
            pl.BlockSpec((BM, K), lambda i: (i, 0)),
            pl.BlockSpec((K, Nn), lambda i: (0, 0)),
            pl.BlockSpec((1, Nn), lambda i: (0, 0)),
        ],
        out_specs=pl.BlockSpec((BM, Nn), lambda i: (i, 0)),
        out_shape=_SDS((M, Nn), _f32),
    )(x, w, b.reshape(1, Nn))


def _pass_a(rows_l, rows_r, ee, att_flat, eot):
    def body(l_ref, r_ref, e_ref, att_ref, eot_ref, a_ref, m_ref):
        i = pl.program_id(0)
        m = l_ref[...] + r_ref[...] + e_ref[...]
        m = jnp.where(m > 0, m, 0.2 * m) * att_ref[...]
        a8 = jnp.dot(m, eot_ref[...], preferred_element_type=_f32)
        a_ref[...] = a8
        bmax = jnp.broadcast_to(jnp.max(a8, axis=0, keepdims=True), (8, 8))

        @pl.when(i == 0)
        def _():
            m_ref[...] = bmax

        @pl.when(i > 0)
        def _():
            m_ref[...] = jnp.maximum(m_ref[...], bmax)

    return pl.pallas_call(
        body,
        grid=(EPAD // BM,),
        in_specs=[
            pl.BlockSpec((BM, D), lambda i: (i, 0)),
            pl.BlockSpec((BM, D), lambda i: (i, 0)),
            pl.BlockSpec((BM, D), lambda i: (i, 0)),
            pl.BlockSpec((1, D), lambda i: (0, 0)),
            pl.BlockSpec((D, 8), lambda i: (0, 0)),
        ],
        out_specs=[
            pl.BlockSpec((BM, 8), lambda i: (i, 0)),
            pl.BlockSpec((8, 8), lambda i: (0, 0)),
        ],
        out_shape=[_SDS((EPAD, 8), _f32), _SDS((8, 8), _f32)],
    )(rows_l, rows_r, ee, att_flat, eot)


def _cumsum_layer(a8, m8, rows_l, eo, eo2, tri):
    def body(a_ref, m_ref, l_ref, eo_ref, eo2_ref, t_ref, o_ref, carry):
        i = pl.program_id(0)

        @pl.when(i == 0)
        def _():
            o_ref[...] = jnp.zeros((CB, PW), _f32)
            carry[...] = jnp.zeros((8, PW), _f32)

        @pl.when(i > 0)
        def _():
            p8 = jnp.exp(a_ref[...] - m_ref[0:1, :])
            v = jnp.dot(p8, eo_ref[...], preferred_element_type=_f32) * l_ref[...]
            pp = jnp.dot(p8, eo2_ref[...], preferred_element_type=_f32)
            blk = jnp.concatenate([v, pp], axis=1)
            cum = jnp.dot(t_ref[...], blk, preferred_element_type=_f32)
            cum = cum + carry[0:1, :]
            o_ref[...] = cum
            carry[...] = jnp.broadcast_to(cum[CB - 1:CB, :], (8, PW))

    prev = lambda i: (jnp.maximum(i - 1, 0), 0)
    return pl.pallas_call(
        body,
        grid=(EPAD // CB + 1,),
        in_specs=[
            pl.BlockSpec((CB, 8), prev),
            pl.BlockSpec((8, 8), lambda i: (0, 0)),
            pl.BlockSpec((CB, D), prev),
            pl.BlockSpec((8, D), lambda i: (0, 0)),
            pl.BlockSpec((8, 128), lambda i: (0, 0)),
            pl.BlockSpec((CB, CB), lambda i: (0, 0)),
        ],
        out_specs=pl.BlockSpec((CB, PW), lambda i: (i, 0)),
        out_shape=_SDS((EPAD + CB, PW), _f32),
        scratch_shapes=[pltpu.VMEM((8, PW), _f32)],
    )(a8, m8, rows_l, eo, eo2, tri)


def _cumsum_pre(ea_s, mask16, tri):
    def body(v_ref, k_ref, t_ref, o_ref, carry):
        i = pl.program_id(0)

        @pl.when(i == 0)
        def _():
            o_ref[...] = jnp.zeros((CB, PW), _f32)
            carry[...] = jnp.zeros((8, PW), _f32)

        @pl.when(i > 0)
        def _():
            blk = jnp.concatenate([v_ref[...], k_ref[...]], axis=1)
            cum = jnp.dot(t_ref[...], blk, preferred_element_type=_f32)
            cum = cum + carry[0:1, :]
            o_ref[...] = cum
            carry[...] = jnp.broadcast_to(cum[CB - 1:CB, :], (8, PW))

    prev = lambda i: (jnp.maximum(i - 1, 0), 0)
    return pl.pallas_call(
        body,
        grid=(EPAD // CB + 1,),
        in_specs=[
            pl.BlockSpec((CB, D), prev),
            pl.BlockSpec((CB, 128), prev),
            pl.BlockSpec((CB, CB), lambda i: (0, 0)),
        ],
        out_specs=pl.BlockSpec((CB, PW), lambda i: (i, 0)),
        out_shape=_SDS((EPAD + CB, PW), _f32),
        scratch_shapes=[pltpu.VMEM((8, PW), _f32)],
    )(ea_s, mask16, tri)


def _combine_layer(hi, lo, hres, gb, eo16):
    def body(hi_ref, lo_ref, hr_ref, gb_ref, eo_ref, out_ref):
        diff = hi_ref[...] - lo_ref[...]
        num = diff[:, :D]
        den = jnp.dot(diff[:, D:PW], eo_ref[...], preferred_element_type=_f32)
        val = jnp.where(den > 0, num / den, 0.0) + gb_ref[...] + hr_ref[...]
        out_ref[...] = _elu(val)

    return pl.pallas_call(
        body,
        grid=(NPAD // BM,),
        in_specs=[
            pl.BlockSpec((BM, PW), lambda i: (i, 0)),
            pl.BlockSpec((BM, PW), lambda i: (i, 0)),
            pl.BlockSpec((BM, D), lambda i: (i, 0)),
            pl.BlockSpec((1, D), lambda i: (0, 0)),
            pl.BlockSpec((128, D), lambda i: (0, 0)),
        ],
        out_specs=pl.BlockSpec((BM, D), lambda i: (i, 0)),
        out_shape=_SDS((NPAD, D), _f32),
    )(hi, lo, hres, gb, eo16)


def _combine_pre(hi, lo):
    def body(hi_ref, lo_ref, out_ref):
        diff = hi_ref[...] - lo_ref[...]
        cnt = jnp.maximum(diff[:, D:D + 1], 1.0)
        out_ref[...] = diff[:, :D] / cnt

    return pl.pallas_call(
        body,
        grid=(NPAD // BM,),
        in_specs=[
            pl.BlockSpec((BM, PW), lambda i: (i, 0)),
            pl.BlockSpec((BM, PW), lambda i: (i, 0)),
        ],
        out_specs=pl.BlockSpec((BM, D), lambda i: (i, 0)),
        out_shape=_SDS((NPAD, D), _f32),
    )(hi, lo)


def _pool(h):
    def body(h_ref, acc_ref):
        i = pl.program_id(0)
        rowid = lax.broadcasted_iota(jnp.int32, (BM, D), 0) + i * BM
        part = jnp.sum(jnp.where(rowid < N, h_ref[...], 0.0), axis=0,
                       keepdims=True)
        part = jnp.broadcast_to(part, (8, D))

        @pl.when(i == 0)
        def _():
            acc_ref[...] = part

        @pl.when(i > 0)
        def _():
            acc_ref[...] = acc_ref[...] + part

    return pl.pallas_call(
        body,
        grid=(NPAD // BM,),
        in_specs=[pl.BlockSpec((BM, D), lambda i: (i, 0))],
        out_specs=pl.BlockSpec((8, D), lambda i: (0, 0)),
        out_shape=_SDS((8, D), _f32),
    )(h)


def _mlp(gacc, w1, b1, w2, b2, w3p, b3p):
    def body(g_ref, w1_ref, b1_ref, w2_ref, b2_ref, w3_ref, b3_ref, out_ref):
        g = g_ref[...] * (1.0 / N)
        z = _elu(jnp.dot(g, w1_ref[...], preferred_element_type=_f32)
                 + b1_ref[...])
        z = _elu(jnp.dot(z, w2_ref[...], preferred_element_type=_f32)
                 + b2_ref[...])
        z = jnp.dot(z, w3_ref[...], preferred_element_type=_f32) + b3_ref[...]
        lane = lax.broadcasted_iota(jnp.int32, (8, 1024), 1)
        msk = lane < OUT_DIM
        mx = jnp.max(jnp.where(msk, z, -jnp.inf), axis=-1, keepdims=True)
        ez = jnp.where(msk, jnp.exp(z - mx), 0.0)
        out_ref[...] = ez / jnp.sum(ez, axis=-1, keepdims=True)

    return pl.pallas_call(
        body,
        grid=(1,),
        in_specs=[
            pl.BlockSpec((8, D), lambda i: (0, 0)),
            pl.BlockSpec((D, 2 * D), lambda i: (0, 0)),
            pl.BlockSpec((1, 2 * D), lambda i: (0, 0)),
            pl.BlockSpec((2 * D, D), lambda i: (0, 0)),
            pl.BlockSpec((1, D), lambda i: (0, 0)),
            pl.BlockSpec((D, 1024), lambda i: (0, 0)),
            pl.BlockSpec((1, 1024), lambda i: (0, 0)),
        ],
        out_specs=pl.BlockSpec((8, 1024), lambda i: (0, 0)),
        out_shape=_SDS((8, 1024), _f32),
    )(gacc, w1, b1, w2, b2, w3p, b3p)


def _mesh():
    return plsc.VectorSubcoreMesh(core_axis_name="c", subcore_axis_name="s")


@functools.partial(jax.jit, static_argnames=("rows", "w", "ch"))
def _sc_gather2(ta, tb, ia, ib, rows, w, ch=CH):
    cpw = rows // (NWORK * ch)

    @functools.partial(
        pl.kernel,
        out_type=[_SDS((rows, w), _f32), _SDS((rows, w), _f32)],
        mesh=_mesh(),
        scratch_types=[
            pltpu.VMEM((ch,), jnp.int32),
            pltpu.VMEM((ch, w), _f32),
            pltpu.VMEM((ch,), jnp.int32),
            pltpu.VMEM((ch, w), _f32),
            pltpu.VMEM((ch,), jnp.int32),
            pltpu.VMEM((ch, w), _f32),
            pltpu.VMEM((ch,), jnp.int32),
            pltpu.VMEM((ch, w), _f32),
        ] + [pltpu.SemaphoreType.DMA] * 8,
    )
    def k(ta_h, tb_h, ia_h, ib_h, oa_h, ob_h, ai, av, bi, bv,
          ai2, av2, bi2, bv2, g1, g2, g3, g4, s1, s2, s3, s4):
        c = lax.axis_index("c")
        s = lax.axis_index("s")
        wid = s * 2 + c

        def body(j, carry):
            offa = (wid * cpw + 2 * j) * ch
            offb = offa + ch
            pltpu.sync_copy(ia_h.at[pl.ds(offa, ch)], ai)
            pltpu.sync_copy(ib_h.at[pl.ds(offa, ch)], bi)
            pltpu.sync_copy(ia_h.at[pl.ds(offb, ch)], ai2)
            pltpu.sync_copy(ib_h.at[pl.ds(offb, ch)], bi2)
            cp1 = pltpu.async_copy(ta_h.at[ai], av, g1)
            cp2 = pltpu.async_copy(tb_h.at[bi], bv, g2)
            cp3 = pltpu.async_copy(ta_h.at[ai2], av2, g3)
            cp4 = pltpu.async_copy(tb_h.at[bi2], bv2, g4)
            cp1.wait()
            st1 = pltpu.async_copy(av, oa_h.at[pl.ds(offa, ch)], s1)
            cp2.wait()
            st2 = pltpu.async_copy(bv, ob_h.at[pl.ds(offa, ch)], s2)
            cp3.wait()
            st3 = pltpu.async_copy(av2, oa_h.at[pl.ds(offb, ch)], s3)
            cp4.wait()
            st4 = pltpu.async_copy(bv2, ob_h.at[pl.ds(offb, ch)], s4)
            st1.wait()
            st2.wait()
            st3.wait()
            st4.wait()
            return carry

        lax.fori_loop(0, cpw // 2, body, 0)

    return k(ta, tb, ia, ib)


@functools.partial(jax.jit, static_argnames=("rows", "w"))
def _sc_gather1(ta, ia, rows, w):
    cpw = rows // (NWORK * CH)

    @functools.partial(
        pl.kernel,
        out_type=[_SDS((rows, w), _f32)],
        mesh=_mesh(),
        scratch_types=[
            pltpu.VMEM((CH,), jnp.int32),
            pltpu.VMEM((CH, w), _f32),
            pltpu.VMEM((CH,), jnp.int32),
            pltpu.VMEM((CH, w), _f32),
        ] + [pltpu.SemaphoreType.DMA] * 4,
    )
    def k(ta_h, ia_h, oa_h, ai, av, ai2, av2, g1, g2, s1, s2):
        c = lax.axis_index("c")
        s = lax.axis_index("s")
        wid = s * 2 + c

        def body(j, carry):
            offa = (wid * cpw + 2 * j) * CH
            offb = offa + CH
            pltpu.sync_copy(ia_h.at[pl.ds(offa, CH)], ai)
            pltpu.sync_copy(ia_h.at[pl.ds(offb, CH)], ai2)
            cp1 = pltpu.async_copy(ta_h.at[ai], av, g1)
            cp2 = pltpu.async_copy(ta_h.at[ai2], av2, g2)
            cp1.wait()
            st1 = pltpu.async_copy(av, oa_h.at[pl.ds(offa, CH)], s1)
            cp2.wait()
            st2 = pltpu.async_copy(av2, oa_h.at[pl.ds(offb, CH)], s2)
            st1.wait()
            st2.wait()
            return carry

        lax.fori_loop(0, cpw // 2, body, 0)

    return k(ta, ia)[0]


def kernel(x, edge_index, edge_attr, enc_W, enc_b, eenc_W, eenc_b, Wl, bl,
           Wr, br, We, att, gbias, Wres, bres, hW1, hb1, hW2, hb2, hW3, hb3):
    src = edge_index[0].astype(jnp.int32)
    dst = edge_index[1].astype(jnp.int32)

    eye8 = jnp.eye(8, dtype=_f32)
    eo = jnp.repeat(eye8, C, axis=1)
    eot = eo.T
    eo2 = jnp.concatenate([eye8, jnp.zeros((8, 120), _f32)], axis=1)
    eo16 = jnp.concatenate([eo, jnp.zeros((120, D), _f32)], axis=0)
    tri = jnp.tril(jnp.ones((CB, CB), _f32))

    loop = jnp.arange(N, dtype=jnp.int32)
    d_ext = jnp.full((EPAD,), N, jnp.int32).at[:EF].set(
        jnp.concatenate([dst, loop]))
    s_ext = jnp.zeros((EPAD,), jnp.int32).at[:EF].set(
        jnp.concatenate([src, loop]))
    perm = jnp.argsort(d_ext, stable=True).astype(jnp.int32)
    d_sorted = d_ext[perm]
    s_g = s_ext[perm]
    idx_pre = jnp.minimum(perm, E)
    idx_full = jnp.minimum(perm, EF)
    ptr = jnp.searchsorted(d_sorted, jnp.arange(NPAD + 1, dtype=jnp.int32)
                           ).astype(jnp.int32)
    idx_hi = jnp.full((BPAD,), CB - 1, jnp.int32).at[:NPAD].set(
        ptr[1:] + (CB - 1))
    idx_lo = jnp.full((BPAD,), CB - 1, jnp.int32).at[:NPAD].set(
        ptr[:-1] + (CB - 1))
    mask16 = jnp.broadcast_to((perm < E).astype(_f32)[:, None], (EPAD, 128))

    xp = jnp.zeros((NPAD, D), _f32).at[:N, :34].set(x)
    encWp = jnp.zeros((D, D), _f32).at[:34].set(enc_W)
    h = _mm_act(xp, encWp, enc_b, True)

    EAP = 160768
    eap = jnp.zeros((EAP, 16), _f32).at[:E].set(edge_attr)
    ea = _mm_act(eap, eenc_W, eenc_b, True)[:E]

    tbl_pre = jnp.concatenate([ea, jnp.zeros((8, D), _f32)], axis=0)
    ea_s = _sc_gather1(tbl_pre, idx_pre, EPAD, D)
    p_pre = _cumsum_pre(ea_s, mask16, tri)
    hi, lo = _sc_gather2(p_pre, p_pre, idx_hi, idx_lo, BPAD, PW, 64)
    mean_ea = _combine_pre(hi[:NPAD], lo[:NPAD])

    tbl_full = jnp.concatenate([ea, mean_ea[:N], jnp.zeros((8, D), _f32)],
                               axis=0)
    ea_full_s = _sc_gather1(tbl_full, idx_full, EPAD, D)

    wcat = jnp.concatenate([Wl, Wr, Wres], axis=2)
    bcat = jnp.concatenate([bl, br, bres], axis=1)

    for i in range(L):
        proj = _mm_act(h, wcat[i], bcat[i], False)
        xl = proj[:, 0:D]
        xr = proj[:, D:2 * D]
        hres = proj[:, 2 * D:3 * D]
        ee = _mm_act(ea_full_s, We[i], jnp.zeros((D,), _f32), False)
        rows_l, rows_r = _sc_gather2(xl, xr, s_g, d_sorted, EPAD, D)
        a8, m8 = _pass_a(rows_l, rows_r, ee, att[i].reshape(1, D), eot)
        ptab = _cumsum_layer(a8, m8, rows_l, eo, eo2, tri)
        hi, lo = _sc_gather2(ptab, ptab, idx_hi, idx_lo, BPAD, PW, 64)
        h = _combine_layer(hi[:NPAD], lo[:NPAD], hres,
                           gbias[i].reshape(1, D), eo16)

    gacc = _pool(h)
    w3p = jnp.zeros((D, 1024), _f32).at[:, :OUT_DIM].set(hW3)
    b3p = jnp.zeros((1, 1024), _f32).at[0, :OUT_DIM].set(hb3)
    out = _mlp(gacc, hW1, hb1.reshape(1, 2 * D), hW2, hb2.reshape(1, D),
               w3p, b3p)
    return out[0:1, :OUT_DIM]


# speedup vs baseline: 1.0311x; 1.0311x over previous
import functools

import jax
import jax.numpy as jnp
from jax import lax
from jax.experimental import pallas as pl
from jax.experimental.pallas import tpu as pltpu
from jax.experimental.pallas import tpu_sc as plsc

N = 10000
E = 160000
L = 4
H = 8
C = 16
D = 128
OUT_DIM = 1000

NPAD = 10240
EF = E + N
EPAD = 172032
BPAD = 12288
CH = 128
NWORK = 32
BM = 1024
CB = 1024
PW = 256

_f32 = jnp.float32
_SDS = jax.ShapeDtypeStruct


def _elu(v):
    return jnp.where(v > 0, v, jnp.exp(jnp.minimum(v, 0.0)) - 1.0)


def _mm_act(x, w, b, act):
    M, K = x.shape
    Nn = w.shape[1]

    def body(x_ref, w_ref, b_ref, o_ref):
        acc = jnp.dot(x_ref[...], w_ref[...], preferred_element_type=_f32)
        acc = acc + b_ref[...]
        o_ref[...] = _elu(acc) if act else acc

    return pl.pallas_call(
        body,
        grid=(M // BM,),
        in_specs=[
            pl.BlockSpec((BM, K), lambda i: (i, 0)),
            pl.BlockSpec((K, Nn), lambda i: (0, 0)),
            pl.BlockSpec((1, Nn), lambda i: (0, 0)),
        ],
        out_specs=pl.BlockSpec((BM, Nn), lambda i: (i, 0)),
        out_shape=_SDS((M, Nn), _f32),
    )(x, w, b.reshape(1, Nn))


def _pass_a(rows_l, rows_r, ee, att_flat, eot):
    def body(l_ref, r_ref, e_ref, att_ref, eot_ref, a_ref, m_ref):
        i = pl.program_id(0)
        m = l_ref[...] + r_ref[...] + e_ref[...]
        m = jnp.where(m > 0, m, 0.2 * m) * att_ref[...]
        a8 = jnp.dot(m, eot_ref[...], preferred_element_type=_f32)
        a_ref[...] = a8
        bmax = jnp.broadcast_to(jnp.max(a8, axis=0, keepdims=True), (8, 8))

        @pl.when(i == 0)
        def _():
            m_ref[...] = bmax

        @pl.when(i > 0)
        def _():
            m_ref[...] = jnp.maximum(m_ref[...], bmax)

    return pl.pallas_call(
        body,
        grid=(EPAD // BM,),
        in_specs=[
            pl.BlockSpec((BM, D), lambda i: (i, 0)),
            pl.BlockSpec((BM, D), lambda i: (i, 0)),
            pl.BlockSpec((BM, D), lambda i: (i, 0)),
            pl.BlockSpec((1, D), lambda i: (0, 0)),
            pl.BlockSpec((D, 8), lambda i: (0, 0)),
        ],
        out_specs=[
            pl.BlockSpec((BM, 8), lambda i: (i, 0)),
            pl.BlockSpec((8, 8), lambda i: (0, 0)),
        ],
        out_shape=[_SDS((EPAD, 8), _f32), _SDS((8, 8), _f32)],
    )(rows_l, rows_r, ee, att_flat, eot)


def _cumsum_layer(a8, m8, rows_l, eo, eo2, tri):
    def body(a_ref, m_ref, l_ref, eo_ref, eo2_ref, t_ref, o_ref, carry):
        i = pl.program_id(0)

        @pl.when(i == 0)
        def _():
            o_ref[...] = jnp.zeros((CB, PW), _f32)
            carry[...] = jnp.zeros((8, PW), _f32)

        @pl.when(i > 0)
        def _():
            p8 = jnp.exp(a_ref[...] - m_ref[0:1, :])
            v = jnp.dot(p8, eo_ref[...], preferred_element_type=_f32) * l_ref[...]
            pp = jnp.dot(p8, eo2_ref[...], preferred_element_type=_f32)
            blk = jnp.concatenate([v, pp], axis=1)
            cum = jnp.dot(t_ref[...], blk, preferred_element_type=_f32)
            cum = cum + carry[0:1, :]
            o_ref[...] = cum
            carry[...] = jnp.broadcast_to(cum[CB - 1:CB, :], (8, PW))

    prev = lambda i: (jnp.maximum(i - 1, 0), 0)
    return pl.pallas_call(
        body,
        grid=(EPAD // CB + 1,),
        in_specs=[
            pl.BlockSpec((CB, 8), prev),
            pl.BlockSpec((8, 8), lambda i: (0, 0)),
            pl.BlockSpec((CB, D), prev),
            pl.BlockSpec((8, D), lambda i: (0, 0)),
            pl.BlockSpec((8, 128), lambda i: (0, 0)),
            pl.BlockSpec((CB, CB), lambda i: (0, 0)),
        ],
        out_specs=pl.BlockSpec((CB, PW), lambda i: (i, 0)),
        out_shape=_SDS((EPAD + CB, PW), _f32),
        scratch_shapes=[pltpu.VMEM((8, PW), _f32)],
    )(a8, m8, rows_l, eo, eo2, tri)


def _cumsum_pre(ea_s, mask16, tri):
    def body(v_ref, k_ref, t_ref, o_ref, carry):
        i = pl.program_id(0)

        @pl.when(i == 0)
        def _():
            o_ref[...] = jnp.zeros((CB, PW), _f32)
            carry[...] = jnp.zeros((8, PW), _f32)

        @pl.when(i > 0)
        def _():
            blk = jnp.concatenate([v_ref[...], k_ref[...]], axis=1)
            cum = jnp.dot(t_ref[...], blk, preferred_element_type=_f32)
            cum = cum + carry[0:1, :]
            o_ref[...] = cum
            carry[...] = jnp.broadcast_to(cum[CB - 1:CB, :], (8, PW))

    prev = lambda i: (jnp.maximum(i - 1, 0), 0)
    return pl.pallas_call(
        body,
        grid=(EPAD // CB + 1,),
        in_specs=[
            pl.BlockSpec((CB, D), prev),
            pl.BlockSpec((CB, 128), prev),
            pl.BlockSpec((CB, CB), lambda i: (0, 0)),
        ],
        out_specs=pl.BlockSpec((CB, PW), lambda i: (i, 0)),
        out_shape=_SDS((EPAD + CB, PW), _f32),
        scratch_shapes=[pltpu.VMEM((8, PW), _f32)],
    )(ea_s, mask16, tri)


def _combine_layer(hi, lo, hres, gb, eo16):
    def body(hi_ref, lo_ref, hr_ref, gb_ref, eo_ref, out_ref):
        diff = hi_ref[...] - lo_ref[...]
        num = diff[:, :D]
        den = jnp.dot(diff[:, D:PW], eo_ref[...], preferred_element_type=_f32)
        val = jnp.where(den > 0, num / den, 0.0) + gb_ref[...] + hr_ref[...]
        out_ref[...] = _elu(val)

    return pl.pallas_call(
        body,
        grid=(NPAD // BM,),
        in_specs=[
            pl.BlockSpec((BM, PW), lambda i: (i, 0)),
            pl.BlockSpec((BM, PW), lambda i: (i, 0)),
            pl.BlockSpec((BM, D), lambda i: (i, 0)),
            pl.BlockSpec((1, D), lambda i: (0, 0)),
            pl.BlockSpec((128, D), lambda i: (0, 0)),
        ],
        out_specs=pl.BlockSpec((BM, D), lambda i: (i, 0)),
        out_shape=_SDS((NPAD, D), _f32),
    )(hi, lo, hres, gb, eo16)


def _combine_pre(hi, lo):
    def body(hi_ref, lo_ref, out_ref):
        diff = hi_ref[...] - lo_ref[...]
        cnt = jnp.maximum(diff[:, D:D + 1], 1.0)
        out_ref[...] = diff[:, :D] / cnt

    return pl.pallas_call(
        body,
        grid=(NPAD // BM,),
        in_specs=[
            pl.BlockSpec((BM, PW), lambda i: (i, 0)),
            pl.BlockSpec((BM, PW), lambda i: (i, 0)),
        ],
        out_specs=pl.BlockSpec((BM, D), lambda i: (i, 0)),
        out_shape=_SDS((NPAD, D), _f32),
    )(hi, lo)


def _pool(h):
    def body(h_ref, acc_ref):
        i = pl.program_id(0)
        rowid = lax.broadcasted_iota(jnp.int32, (BM, D), 0) + i * BM
        part = jnp.sum(jnp.where(rowid < N, h_ref[...], 0.0), axis=0,
                       keepdims=True)
        part = jnp.broadcast_to(part, (8, D))

        @pl.when(i == 0)
        def _():
            acc_ref[...] = part

        @pl.when(i > 0)
        def _():
            acc_ref[...] = acc_ref[...] + part

    return pl.pallas_call(
        body,
        grid=(NPAD // BM,),
        in_specs=[pl.BlockSpec((BM, D), lambda i: (i, 0))],
        out_specs=pl.BlockSpec((8, D), lambda i: (0, 0)),
        out_shape=_SDS((8, D), _f32),
    )(h)


def _mlp(gacc, w1, b1, w2, b2, w3p, b3p):
    def body(g_ref, w1_ref, b1_ref, w2_ref, b2_ref, w3_ref, b3_ref, out_ref):
        g = g_ref[...] * (1.0 / N)
        z = _elu(jnp.dot(g, w1_ref[...], preferred_element_type=_f32)
                 + b1_ref[...])
        z = _elu(jnp.dot(z, w2_ref[...], preferred_element_type=_f32)
                 + b2_ref[...])
        z = jnp.dot(z, w3_ref[...], preferred_element_type=_f32) + b3_ref[...]
        lane = lax.broadcasted_iota(jnp.int32, (8, 1024), 1)
        msk = lane < OUT_DIM
        mx = jnp.max(jnp.where(msk, z, -jnp.inf), axis=-1, keepdims=True)
        ez = jnp.where(msk, jnp.exp(z - mx), 0.0)
        out_ref[...] = ez / jnp.sum(ez, axis=-1, keepdims=True)

    return pl.pallas_call(
        body,
        grid=(1,),
        in_specs=[
            pl.BlockSpec((8, D), lambda i: (0, 0)),
            pl.BlockSpec((D, 2 * D), lambda i: (0, 0)),
            pl.BlockSpec((1, 2 * D), lambda i: (0, 0)),
            pl.BlockSpec((2 * D, D), lambda i: (0, 0)),
            pl.BlockSpec((1, D), lambda i: (0, 0)),
            pl.BlockSpec((D, 1024), lambda i: (0, 0)),
            pl.BlockSpec((1, 1024), lambda i: (0, 0)),
        ],
        out_specs=pl.BlockSpec((8, 1024), lambda i: (0, 0)),
        out_shape=_SDS((8, 1024), _f32),
    )(gacc, w1, b1, w2, b2, w3p, b3p)


def _mesh():
    return plsc.VectorSubcoreMesh(core_axis_name="c", subcore_axis_name="s")


@functools.partial(jax.jit, static_argnames=("rows", "w", "ch"))
def _sc_gather2(ta, tb, ia, ib, rows, w, ch=CH):
    cpw = rows // (NWORK * ch)

    @functools.partial(
        pl.kernel,
        out_type=[_SDS((rows, w), _f32), _SDS((rows, w), _f32)],
        mesh=_mesh(),
        scratch_types=[
            pltpu.VMEM((ch,), jnp.int32),
            pltpu.VMEM((ch, w), _f32),
            pltpu.VMEM((ch,), jnp.int32),
            pltpu.VMEM((ch, w), _f32),
            pltpu.VMEM((ch,), jnp.int32),
            pltpu.VMEM((ch, w), _f32),
            pltpu.VMEM((ch,), jnp.int32),
            pltpu.VMEM((ch, w), _f32),
        ] + [pltpu.SemaphoreType.DMA] * 8,
    )
    def k(ta_h, tb_h, ia_h, ib_h, oa_h, ob_h, ai, av, bi, bv,
          ai2, av2, bi2, bv2, g1, g2, g3, g4, s1, s2, s3, s4):
        c = lax.axis_index("c")
        s = lax.axis_index("s")
        wid = s * 2 + c

        def body(j, carry):
            offa = (wid * cpw + 2 * j) * ch
            offb = offa + ch
            pltpu.sync_copy(ia_h.at[pl.ds(offa, ch)], ai)
            pltpu.sync_copy(ib_h.at[pl.ds(offa, ch)], bi)
            pltpu.sync_copy(ia_h.at[pl.ds(offb, ch)], ai2)
            pltpu.sync_copy(ib_h.at[pl.ds(offb, ch)], bi2)
            cp1 = pltpu.async_copy(ta_h.at[ai], av, g1)
            cp2 = pltpu.async_copy(tb_h.at[bi], bv, g2)
            cp3 = pltpu.async_copy(ta_h.at[ai2], av2, g3)
            cp4 = pltpu.async_copy(tb_h.at[bi2], bv2, g4)
            cp1.wait()
            st1 = pltpu.async_copy(av, oa_h.at[pl.ds(offa, ch)], s1)
            cp2.wait()
            st2 = pltpu.async_copy(bv, ob_h.at[pl.ds(offa, ch)], s2)
            cp3.wait()
            st3 = pltpu.async_copy(av2, oa_h.at[pl.ds(offb, ch)], s3)
            cp4.wait()
            st4 = pltpu.async_copy(bv2, ob_h.at[pl.ds(offb, ch)], s4)
            st1.wait()
            st2.wait()
            st3.wait()
            st4.wait()
            return carry

        lax.fori_loop(0, cpw // 2, body, 0)

    return k(ta, tb, ia, ib)


@functools.partial(jax.jit, static_argnames=("rows", "w"))
def _sc_gather1(ta, ia, rows, w):
    cpw = rows // (NWORK * CH)

    @functools.partial(
        pl.kernel,
        out_type=[_SDS((rows, w), _f32)],
        mesh=_mesh(),
        scratch_types=[
            pltpu.VMEM((CH,), jnp.int32),
            pltpu.VMEM((CH, w), _f32),
            pltpu.VMEM((CH,), jnp.int32),
            pltpu.VMEM((CH, w), _f32),
        ] + [pltpu.SemaphoreType.DMA] * 4,
    )
    def k(ta_h, ia_h, oa_h, ai, av, ai2, av2, g1, g2, s1, s2):
        c = lax.axis_index("c")
        s = lax.axis_index("s")
        wid = s * 2 + c

        def body(j, carry):
            offa = (wid * cpw + 2 * j) * CH
            offb = offa + CH
            pltpu.sync_copy(ia_h.at[pl.ds(offa, CH)], ai)
            pltpu.sync_copy(ia_h.at[pl.ds(offb, CH)], ai2)
            cp1 = pltpu.async_copy(ta_h.at[ai], av, g1)
            cp2 = pltpu.async_copy(ta_h.at[ai2], av2, g2)
            cp1.wait()
            st1 = pltpu.async_copy(av, oa_h.at[pl.ds(offa, CH)], s1)
            cp2.wait()
            st2 = pltpu.async_copy(av2, oa_h.at[pl.ds(offb, CH)], s2)
            st1.wait()
            st2.wait()
            return carry

        lax.fori_loop(0, cpw // 2, body, 0)

    return k(ta, ia)[0]


def kernel(x, edge_index, edge_attr, enc_W, enc_b, eenc_W, eenc_b, Wl, bl,
           Wr, br, We, att, gbias, Wres, bres, hW1, hb1, hW2, hb2, hW3, hb3):
    src = edge_index[0].astype(jnp.int32)
    dst = edge_index[1].astype(jnp.int32)

    eye8 = jnp.eye(8, dtype=_f32)
    eo = jnp.repeat(eye8, C, axis=1)
    eot = eo.T
    eo2 = jnp.concatenate([eye8, jnp.zeros((8, 120), _f32)], axis=1)
    eo16 = jnp.concatenate([eo, jnp.zeros((120, D), _f32)], axis=0)
    tri = jnp.tril(jnp.ones((CB, CB), _f32))

    loop = jnp.arange(N, dtype=jnp.int32)
    d_ext = jnp.full((EPAD,), N, jnp.int32).at[:EF].set(
        jnp.concatenate([dst, loop]))
    s_ext = jnp.zeros((EPAD,), jnp.int32).at[:EF].set(
        jnp.concatenate([src, loop]))
    perm = jnp.argsort(d_ext, stable=True).astype(jnp.int32)
    d_sorted = d_ext[perm]
    s_g = s_ext[perm]
    idx_pre = jnp.minimum(perm, E)
    idx_full = jnp.minimum(perm, EF)
    ptr = jnp.searchsorted(d_sorted, jnp.arange(NPAD + 1, dtype=jnp.int32)
                           ).astype(jnp.int32)
    idx_hi = jnp.full((BPAD,), CB - 1, jnp.int32).at[:NPAD].set(
        ptr[1:] + (CB - 1))
    idx_lo = jnp.full((BPAD,), CB - 1, jnp.int32).at[:NPAD].set(
        ptr[:-1] + (CB - 1))
    mask16 = jnp.broadcast_to((perm < E).astype(_f32)[:, None], (EPAD, 128))

    xp = jnp.zeros((NPAD, D), _f32).at[:N, :34].set(x)
    encWp = jnp.zeros((D, D), _f32).at[:34].set(enc_W)
    h = _mm_act(xp, encWp, enc_b, True)

    EAP = 160768
    eap = jnp.zeros((EAP, 16), _f32).at[:E].set(edge_attr)
    ea = _mm_act(eap, eenc_W, eenc_b, True)[:E]

    tbl_pre = jnp.concatenate([ea, jnp.zeros((8, D), _f32)], axis=0)
    ea_s = _sc_gather1(tbl_pre, idx_pre, EPAD, D)
    p_pre = _cumsum_pre(ea_s, mask16, tri)
    hi, lo = _sc_gather2(p_pre, p_pre, idx_hi, idx_lo, BPAD, PW, 64)
    mean_ea = _combine_pre(hi[:NPAD], lo[:NPAD])

    tbl_full = jnp.concatenate([ea, mean_ea[:N], jnp.zeros((8, D), _f32)],
                               axis=0)
    ea_full_s = _sc_gather1(tbl_full, idx_full, EPAD, D)

    wcat = jnp.concatenate([Wl, Wr, Wres], axis=2)
    bcat = jnp.concatenate([bl, br, bres], axis=1)

    for i in range(L):
        proj = _mm_act(h, wcat[i], bcat[i], False)
        xl = proj[:, 0:D]
        xr = proj[:, D:2 * D]
        hres = proj[:, 2 * D:3 * D]
        ee = _mm_act(ea_full_s, We[i], jnp.zeros((D,), _f32), False)
        rows_l, rows_r = _sc_gather2(xl, xr, s_g, d_sorted, EPAD, D)
        a8, m8 = _pass_a(rows_l, rows_r, ee, att[i].reshape(1, D), eot)
        ptab = _cumsum_layer(a8, m8, rows_l, eo, eo2, tri)
        hi, lo = _sc_gather2(ptab, ptab, idx_hi, idx_lo, BPAD, PW, 64)
        h = _combine_layer(hi[:NPAD], lo[:NPAD], hres,
                           gbias[i].reshape(1, D), eo16)

    gacc = _pool(h)
    w3p = jnp.zeros((D, 1024), _f32).at[:, :OUT_DIM].set(hW3)
    b3p = jnp.zeros((1, 1024), _f32).at[0, :OUT_DIM].set(hb3)
    out = _mlp(gacc, hW1, hb1.reshape(1, 2 * D), hW2, hb2.reshape(1, D),
               w3p, b3p)
    return out[0:1, :OUT_DIM]


# speedup vs baseline: 1.9160x; 1.8582x over previous
import functools

import jax
import jax.numpy as jnp
from jax import lax
from jax.experimental import pallas as pl
from jax.experimental.pallas import tpu as pltpu
from jax.experimental.pallas import tpu_sc as plsc

N = 10000
E = 160000
L = 4
H = 8
C = 16
D = 128
OUT_DIM = 1000

NPAD = 10240
EF = E + N
EPAD = 172032
BPAD = 12288
CH = 128
NWORK = 32
BM = 1024
CB = 1024
PW = 256

_f32 = jnp.float32
_SDS = jax.ShapeDtypeStruct


def _elu(v):
    return jnp.where(v > 0, v, jnp.exp(jnp.minimum(v, 0.0)) - 1.0)


def _mm_act(x, w, b, act):
    M, K = x.shape
    Nn = w.shape[1]

    def body(x_ref, w_ref, b_ref, o_ref):
        acc = jnp.dot(x_ref[...], w_ref[...], preferred_element_type=_f32)
        acc = acc + b_ref[...]
        o_ref[...] = _elu(acc) if act else acc

    return pl.pallas_call(
        body,
        grid=(M // BM,),
        in_specs=[
            pl.BlockSpec((BM, K), lambda i: (i, 0)),
            pl.BlockSpec((K, Nn), lambda i: (0, 0)),
            pl.BlockSpec((1, Nn), lambda i: (0, 0)),
        ],
        out_specs=pl.BlockSpec((BM, Nn), lambda i: (i, 0)),
        out_shape=_SDS((M, Nn), _f32),
    )(x, w, b.reshape(1, Nn))


def _pass_a(rows_l, rows_r, ee, att_flat, eot):
    def body(l_ref, r_ref, e_ref, att_ref, eot_ref, a_ref, m_ref):
        i = pl.program_id(0)
        m = l_ref[...] + r_ref[...] + e_ref[...]
        m = jnp.where(m > 0, m, 0.2 * m) * att_ref[...]
        a8 = jnp.dot(m, eot_ref[...], preferred_element_type=_f32)
        a_ref[...] = a8
        bmax = jnp.broadcast_to(jnp.max(a8, axis=0, keepdims=True), (8, 8))

        @pl.when(i == 0)
        def _():
            m_ref[...] = bmax

        @pl.when(i > 0)
        def _():
            m_ref[...] = jnp.maximum(m_ref[...], bmax)

    return pl.pallas_call(
        body,
        grid=(EPAD // BM,),
        in_specs=[
            pl.BlockSpec((BM, D), lambda i: (i, 0)),
            pl.BlockSpec((BM, D), lambda i: (i, 0)),
            pl.BlockSpec((BM, D), lambda i: (i, 0)),
            pl.BlockSpec((1, D), lambda i: (0, 0)),
            pl.BlockSpec((D, 8), lambda i: (0, 0)),
        ],
        out_specs=[
            pl.BlockSpec((BM, 8), lambda i: (i, 0)),
            pl.BlockSpec((8, 8), lambda i: (0, 0)),
        ],
        out_shape=[_SDS((EPAD, 8), _f32), _SDS((8, 8), _f32)],
    )(rows_l, rows_r, ee, att_flat, eot)


def _cumsum_layer(a8, m8, rows_l, eo, eo2, tri):
    def body(a_ref, m_ref, l_ref, eo_ref, eo2_ref, t_ref, o_ref, carry):
        i = pl.program_id(0)

        @pl.when(i == 0)
        def _():
            o_ref[...] = jnp.zeros((CB, PW), _f32)
            carry[...] = jnp.zeros((8, PW), _f32)

        @pl.when(i > 0)
        def _():
            p8 = jnp.exp(a_ref[...] - m_ref[0:1, :])
            v = jnp.dot(p8, eo_ref[...], preferred_element_type=_f32) * l_ref[...]
            pp = jnp.dot(p8, eo2_ref[...], preferred_element_type=_f32)
            blk = jnp.concatenate([v, pp], axis=1)
            cum = jnp.dot(t_ref[...], blk, preferred_element_type=_f32)
            cum = cum + carry[0:1, :]
            o_ref[...] = cum
            carry[...] = jnp.broadcast_to(cum[CB - 1:CB, :], (8, PW))

    prev = lambda i: (jnp.maximum(i - 1, 0), 0)
    return pl.pallas_call(
        body,
        grid=(EPAD // CB + 1,),
        in_specs=[
            pl.BlockSpec((CB, 8), prev),
            pl.BlockSpec((8, 8), lambda i: (0, 0)),
            pl.BlockSpec((CB, D), prev),
            pl.BlockSpec((8, D), lambda i: (0, 0)),
            pl.BlockSpec((8, 128), lambda i: (0, 0)),
            pl.BlockSpec((CB, CB), lambda i: (0, 0)),
        ],
        out_specs=pl.BlockSpec((CB, PW), lambda i: (i, 0)),
        out_shape=_SDS((EPAD + CB, PW), _f32),
        scratch_shapes=[pltpu.VMEM((8, PW), _f32)],
    )(a8, m8, rows_l, eo, eo2, tri)


def _cumsum_pre(ea_s, mask16, tri):
    def body(v_ref, k_ref, t_ref, o_ref, carry):
        i = pl.program_id(0)

        @pl.when(i == 0)
        def _():
            o_ref[...] = jnp.zeros((CB, PW), _f32)
            carry[...] = jnp.zeros((8, PW), _f32)

        @pl.when(i > 0)
        def _():
            blk = jnp.concatenate([v_ref[...], k_ref[...]], axis=1)
            cum = jnp.dot(t_ref[...], blk, preferred_element_type=_f32)
            cum = cum + carry[0:1, :]
            o_ref[...] = cum
            carry[...] = jnp.broadcast_to(cum[CB - 1:CB, :], (8, PW))

    prev = lambda i: (jnp.maximum(i - 1, 0), 0)
    return pl.pallas_call(
        body,
        grid=(EPAD // CB + 1,),
        in_specs=[
            pl.BlockSpec((CB, D), prev),
            pl.BlockSpec((CB, 128), prev),
            pl.BlockSpec((CB, CB), lambda i: (0, 0)),
        ],
        out_specs=pl.BlockSpec((CB, PW), lambda i: (i, 0)),
        out_shape=_SDS((EPAD + CB, PW), _f32),
        scratch_shapes=[pltpu.VMEM((8, PW), _f32)],
    )(ea_s, mask16, tri)


def _combine_layer(hi, lo, hres, gb, eo16):
    def body(hi_ref, lo_ref, hr_ref, gb_ref, eo_ref, out_ref):
        diff = hi_ref[...] - lo_ref[...]
        num = diff[:, :D]
        den = jnp.dot(diff[:, D:PW], eo_ref[...], preferred_element_type=_f32)
        val = jnp.where(den > 0, num / den, 0.0) + gb_ref[...] + hr_ref[...]
        out_ref[...] = _elu(val)

    return pl.pallas_call(
        body,
        grid=(NPAD // BM,),
        in_specs=[
            pl.BlockSpec((BM, PW), lambda i: (i, 0)),
            pl.BlockSpec((BM, PW), lambda i: (i, 0)),
            pl.BlockSpec((BM, D), lambda i: (i, 0)),
            pl.BlockSpec((1, D), lambda i: (0, 0)),
            pl.BlockSpec((128, D), lambda i: (0, 0)),
        ],
        out_specs=pl.BlockSpec((BM, D), lambda i: (i, 0)),
        out_shape=_SDS((NPAD, D), _f32),
    )(hi, lo, hres, gb, eo16)


def _combine_pre(hi, lo):
    def body(hi_ref, lo_ref, out_ref):
        diff = hi_ref[...] - lo_ref[...]
        cnt = jnp.maximum(diff[:, D:D + 1], 1.0)
        out_ref[...] = diff[:, :D] / cnt

    return pl.pallas_call(
        body,
        grid=(NPAD // BM,),
        in_specs=[
            pl.BlockSpec((BM, PW), lambda i: (i, 0)),
            pl.BlockSpec((BM, PW), lambda i: (i, 0)),
        ],
        out_specs=pl.BlockSpec((BM, D), lambda i: (i, 0)),
        out_shape=_SDS((NPAD, D), _f32),
    )(hi, lo)


def _pool(h):
    def body(h_ref, acc_ref):
        i = pl.program_id(0)
        rowid = lax.broadcasted_iota(jnp.int32, (BM, D), 0) + i * BM
        part = jnp.sum(jnp.where(rowid < N, h_ref[...], 0.0), axis=0,
                       keepdims=True)
        part = jnp.broadcast_to(part, (8, D))

        @pl.when(i == 0)
        def _():
            acc_ref[...] = part

        @pl.when(i > 0)
        def _():
            acc_ref[...] = acc_ref[...] + part

    return pl.pallas_call(
        body,
        grid=(NPAD // BM,),
        in_specs=[pl.BlockSpec((BM, D), lambda i: (i, 0))],
        out_specs=pl.BlockSpec((8, D), lambda i: (0, 0)),
        out_shape=_SDS((8, D), _f32),
    )(h)


def _mlp(gacc, w1, b1, w2, b2, w3p, b3p):
    def body(g_ref, w1_ref, b1_ref, w2_ref, b2_ref, w3_ref, b3_ref, out_ref):
        g = g_ref[...] * (1.0 / N)
        z = _elu(jnp.dot(g, w1_ref[...], preferred_element_type=_f32)
                 + b1_ref[...])
        z = _elu(jnp.dot(z, w2_ref[...], preferred_element_type=_f32)
                 + b2_ref[...])
        z = jnp.dot(z, w3_ref[...], preferred_element_type=_f32) + b3_ref[...]
        lane = lax.broadcasted_iota(jnp.int32, (8, 1024), 1)
        msk = lane < OUT_DIM
        mx = jnp.max(jnp.where(msk, z, -jnp.inf), axis=-1, keepdims=True)
        ez = jnp.where(msk, jnp.exp(z - mx), 0.0)
        out_ref[...] = ez / jnp.sum(ez, axis=-1, keepdims=True)

    return pl.pallas_call(
        body,
        grid=(1,),
        in_specs=[
            pl.BlockSpec((8, D), lambda i: (0, 0)),
            pl.BlockSpec((D, 2 * D), lambda i: (0, 0)),
            pl.BlockSpec((1, 2 * D), lambda i: (0, 0)),
            pl.BlockSpec((2 * D, D), lambda i: (0, 0)),
            pl.BlockSpec((1, D), lambda i: (0, 0)),
            pl.BlockSpec((D, 1024), lambda i: (0, 0)),
            pl.BlockSpec((1, 1024), lambda i: (0, 0)),
        ],
        out_specs=pl.BlockSpec((8, 1024), lambda i: (0, 0)),
        out_shape=_SDS((8, 1024), _f32),
    )(gacc, w1, b1, w2, b2, w3p, b3p)


def _mesh():
    return plsc.VectorSubcoreMesh(core_axis_name="c", subcore_axis_name="s")


@functools.partial(jax.jit, static_argnames=("rows", "w", "ch"))
def _sc_gather2(ta, tb, ia, ib, rows, w, ch=CH):
    cpw = rows // (NWORK * ch)

    @functools.partial(
        pl.kernel,
        out_type=[_SDS((rows, w), _f32), _SDS((rows, w), _f32)],
        mesh=_mesh(),
        scratch_types=[
            pltpu.VMEM((ch,), jnp.int32),
            pltpu.VMEM((ch, w), _f32),
            pltpu.VMEM((ch,), jnp.int32),
            pltpu.VMEM((ch, w), _f32),
            pltpu.VMEM((ch,), jnp.int32),
            pltpu.VMEM((ch, w), _f32),
            pltpu.VMEM((ch,), jnp.int32),
            pltpu.VMEM((ch, w), _f32),
        ] + [pltpu.SemaphoreType.DMA] * 8,
    )
    def k(ta_h, tb_h, ia_h, ib_h, oa_h, ob_h, ai, av, bi, bv,
          ai2, av2, bi2, bv2, g1, g2, g3, g4, s1, s2, s3, s4):
        c = lax.axis_index("c")
        s = lax.axis_index("s")
        wid = s * 2 + c

        def body(j, carry):
            offa = (wid * cpw + 2 * j) * ch
            offb = offa + ch
            pltpu.sync_copy(ia_h.at[pl.ds(offa, ch)], ai)
            pltpu.sync_copy(ib_h.at[pl.ds(offa, ch)], bi)
            pltpu.sync_copy(ia_h.at[pl.ds(offb, ch)], ai2)
            pltpu.sync_copy(ib_h.at[pl.ds(offb, ch)], bi2)
            cp1 = pltpu.async_copy(ta_h.at[ai], av, g1)
            cp2 = pltpu.async_copy(tb_h.at[bi], bv, g2)
            cp3 = pltpu.async_copy(ta_h.at[ai2], av2, g3)
            cp4 = pltpu.async_copy(tb_h.at[bi2], bv2, g4)
            cp1.wait()
            st1 = pltpu.async_copy(av, oa_h.at[pl.ds(offa, ch)], s1)
            cp2.wait()
            st2 = pltpu.async_copy(bv, ob_h.at[pl.ds(offa, ch)], s2)
            cp3.wait()
            st3 = pltpu.async_copy(av2, oa_h.at[pl.ds(offb, ch)], s3)
            cp4.wait()
            st4 = pltpu.async_copy(bv2, ob_h.at[pl.ds(offb, ch)], s4)
            st1.wait()
            st2.wait()
            st3.wait()
            st4.wait()
            return carry

        lax.fori_loop(0, cpw // 2, body, 0)

    return k(ta, tb, ia, ib)


@functools.partial(jax.jit, static_argnames=("rows", "w"))
def _sc_gather1(ta, ia, rows, w):
    cpw = rows // (NWORK * CH)

    @functools.partial(
        pl.kernel,
        out_type=[_SDS((rows, w), _f32)],
        mesh=_mesh(),
        scratch_types=[
            pltpu.VMEM((CH,), jnp.int32),
            pltpu.VMEM((CH, w), _f32),
            pltpu.VMEM((CH,), jnp.int32),
            pltpu.VMEM((CH, w), _f32),
        ] + [pltpu.SemaphoreType.DMA] * 4,
    )
    def k(ta_h, ia_h, oa_h, ai, av, ai2, av2, g1, g2, s1, s2):
        c = lax.axis_index("c")
        s = lax.axis_index("s")
        wid = s * 2 + c

        def body(j, carry):
            offa = (wid * cpw + 2 * j) * CH
            offb = offa + CH
            pltpu.sync_copy(ia_h.at[pl.ds(offa, CH)], ai)
            pltpu.sync_copy(ia_h.at[pl.ds(offb, CH)], ai2)
            cp1 = pltpu.async_copy(ta_h.at[ai], av, g1)
            cp2 = pltpu.async_copy(ta_h.at[ai2], av2, g2)
            cp1.wait()
            st1 = pltpu.async_copy(av, oa_h.at[pl.ds(offa, CH)], s1)
            cp2.wait()
            st2 = pltpu.async_copy(av2, oa_h.at[pl.ds(offb, CH)], s2)
            st1.wait()
            st2.wait()
            return carry

        lax.fori_loop(0, cpw // 2, body, 0)

    return k(ta, ia)[0]


def _stack_xr2(xr, mee):
    nb = NPAD // BM

    def body(xr_ref, mee_ref, o_ref):
        i = pl.program_id(0)

        @pl.when(i < nb)
        def _():
            o_ref[...] = xr_ref[...]

        @pl.when(i >= nb)
        def _():
            o_ref[...] = xr_ref[...] + mee_ref[...]

    rowblk = lambda i: (lax.rem(i, nb), 0)
    return pl.pallas_call(
        body,
        grid=(2 * nb,),
        in_specs=[
            pl.BlockSpec((BM, D), rowblk),
            pl.BlockSpec((BM, D), rowblk),
        ],
        out_specs=pl.BlockSpec((BM, D), lambda i: (i, 0)),
        out_shape=_SDS((2 * NPAD, D), _f32),
    )(xr, mee)


def kernel(x, edge_index, edge_attr, enc_W, enc_b, eenc_W, eenc_b, Wl, bl,
           Wr, br, We, att, gbias, Wres, bres, hW1, hb1, hW2, hb2, hW3, hb3):
    src = edge_index[0].astype(jnp.int32)
    dst = edge_index[1].astype(jnp.int32)

    eye8 = jnp.eye(8, dtype=_f32)
    eo = jnp.repeat(eye8, C, axis=1)
    eot = eo.T
    eo2 = jnp.concatenate([eye8, jnp.zeros((8, 120), _f32)], axis=1)
    eo16 = jnp.concatenate([eo, jnp.zeros((120, D), _f32)], axis=0)
    tri = jnp.tril(jnp.ones((CB, CB), _f32))

    loop = jnp.arange(N, dtype=jnp.int32)
    d_ext = jnp.full((EPAD,), N, jnp.int32).at[:EF].set(
        jnp.concatenate([dst, loop]))
    s_ext = jnp.zeros((EPAD,), jnp.int32).at[:EF].set(
        jnp.concatenate([src, loop]))
    perm = jnp.argsort(d_ext, stable=True).astype(jnp.int32)
    d_sorted = d_ext[perm]
    s_g = s_ext[perm]
    idx_pre = jnp.minimum(perm, E)
    isloop = (perm >= E) & (perm < EF)
    d2 = d_sorted + isloop.astype(jnp.int32) * NPAD
    ptr = jnp.searchsorted(d_sorted, jnp.arange(NPAD + 1, dtype=jnp.int32)
                           ).astype(jnp.int32)
    idx_hi = jnp.full((BPAD,), CB - 1, jnp.int32).at[:NPAD].set(
        ptr[1:] + (CB - 1))
    idx_lo = jnp.full((BPAD,), CB - 1, jnp.int32).at[:NPAD].set(
        ptr[:-1] + (CB - 1))
    mask16 = jnp.broadcast_to((perm < E).astype(_f32)[:, None], (EPAD, 128))

    xp = jnp.zeros((NPAD, D), _f32).at[:N, :34].set(x)
    encWp = jnp.zeros((D, D), _f32).at[:34].set(enc_W)
    h = _mm_act(xp, encWp, enc_b, True)

    EAP = 160768
    eap = jnp.zeros((EAP, 16), _f32).at[:E].set(edge_attr)
    ea = _mm_act(eap, eenc_W, eenc_b, True)[:E]

    tbl_pre = jnp.concatenate([ea, jnp.zeros((8, D), _f32)], axis=0)
    ea_s = _sc_gather1(tbl_pre, idx_pre, EPAD, D)
    p_pre = _cumsum_pre(ea_s, mask16, tri)
    hi, lo = _sc_gather2(p_pre, p_pre, idx_hi, idx_lo, BPAD, PW, 64)
    mean_ea = _combine_pre(hi[:NPAD], lo[:NPAD])

    wcat = jnp.concatenate([Wl, Wr, Wres], axis=2)
    bcat = jnp.concatenate([bl, br, bres], axis=1)

    for i in range(L):
        proj = _mm_act(h, wcat[i], bcat[i], False)
        xl = proj[:, 0:D]
        xr = proj[:, D:2 * D]
        hres = proj[:, 2 * D:3 * D]
        ee = _mm_act(ea_s, We[i], jnp.zeros((D,), _f32), False)
        mee = _mm_act(mean_ea, We[i], jnp.zeros((D,), _f32), False)
        xr2t = _stack_xr2(xr, mee)
        rows_l, rows_r = _sc_gather2(xl, xr2t, s_g, d2, EPAD, D)
        a8, m8 = _pass_a(rows_l, rows_r, ee, att[i].reshape(1, D), eot)
        ptab = _cumsum_layer(a8, m8, rows_l, eo, eo2, tri)
        hi, lo = _sc_gather2(ptab, ptab, idx_hi, idx_lo, BPAD, PW, 64)
        h = _combine_layer(hi[:NPAD], lo[:NPAD], hres,
                           gbias[i].reshape(1, D), eo16)

    gacc = _pool(h)
    w3p = jnp.zeros((D, 1024), _f32).at[:, :OUT_DIM].set(hW3)
    b3p = jnp.zeros((1, 1024), _f32).at[0, :OUT_DIM].set(hb3)
    out = _mlp(gacc, hW1, hb1.reshape(1, 2 * D), hW2, hb2.reshape(1, D),
               w3p, b3p)
    return out[0:1, :OUT_DIM]
